# baseline (speedup 1.0000x reference)
import functools

import jax
import jax.numpy as jnp
from jax import lax
from jax.experimental import pallas as pl
from jax.experimental.pallas import tpu as pltpu
from jax.experimental.pallas import tpu_sc as plsc

SCALE = 8.0

_info = plsc.get_sparse_core_info()
_NC, _NS, _L = _info.num_cores, _info.num_subcores, _info.num_lanes
_NW = _NC * _NS

_NBUF = 2


@functools.lru_cache(maxsize=None)
def _make_sc_gather(V, D, P, N):
    C = N // _NW
    assert C <= 128 and D % _L == 0 and P % _NBUF == 0
    mesh = plsc.VectorSubcoreMesh(core_axis_name="c", subcore_axis_name="s")

    @functools.partial(
        pl.kernel,
        mesh=mesh,
        compiler_params=pltpu.CompilerParams(
            use_tc_tiling_on_sc=False, needs_layout_passes=False),
        out_type=jax.ShapeDtypeStruct((P * D, N), jnp.float32),
        scratch_types=[
            pltpu.VMEM((P, C), jnp.int32),
            pltpu.VMEM((C, D), jnp.float32),
            pltpu.VMEM((C, D), jnp.float32),
            pltpu.VMEM((D, C), jnp.float32),
            pltpu.VMEM((D, C), jnp.float32),
            pltpu.SemaphoreType.DMA,
            pltpu.SemaphoreType.DMA,
            pltpu.SemaphoreType.DMA,
            pltpu.SemaphoreType.DMA,
        ],
    )
    def k(table_hbm, idx_hbm, out_hbm, idx_v, g0, g1, t0, t1,
          gs0, gs1, os0, os1):
        gbufs, tbufs = (g0, g1), (t0, t1)
        gsems, osems = (gs0, gs1), (os0, os1)
        wid = lax.axis_index("s") * _NC + lax.axis_index("c")
        b0 = wid * C
        pltpu.sync_copy(idx_hbm.at[:, pl.ds(b0, C)], idx_v)

        def fire(buf, sem, p):
            pltpu.async_copy(table_hbm.at[idx_v.at[p]], buf, sem)

        def drain_gather(buf, sem):
            pltpu.make_async_copy(table_hbm.at[idx_v.at[0]], buf, sem).wait()

        def transpose_scale(gbuf, tbuf):
            def body(r, carry):
                cols = jnp.full((_L,), r, jnp.int32)
                for q in range(D // _L):
                    rows = lax.iota(jnp.int32, _L) + q * _L
                    v = gbuf[r, pl.ds(q * _L, _L)]
                    plsc.store_scatter(tbuf, [rows, cols], v * SCALE)
                return carry
            lax.fori_loop(0, C, body, 0)

        def writeback(tbuf, sem, p):
            pltpu.async_copy(
                tbuf, out_hbm.at[pl.ds(p * D, D), pl.ds(b0, C)], sem)

        def drain_out(tbuf, sem):
            pltpu.make_async_copy(
                tbuf, out_hbm.at[pl.ds(0, D), pl.ds(0, C)], sem).wait()

        for b in range(_NBUF):
            fire(gbufs[b], gsems[b], b)

        def outer(g, carry):
            for b in range(_NBUF):
                p = g * _NBUF + b
                drain_gather(gbufs[b], gsems[b])

                @pl.when(g > 0)
                def _():
                    drain_out(tbufs[b], osems[b])

                transpose_scale(gbufs[b], tbufs[b])
                fire(gbufs[b], gsems[b], p + _NBUF)
                writeback(tbufs[b], osems[b], p)
            return carry

        lax.fori_loop(0, P // _NBUF - 1, outer, 0)

        for b in range(_NBUF):
            p = P - _NBUF + b
            drain_gather(gbufs[b], gsems[b])
            drain_out(tbufs[b], osems[b])
            transpose_scale(gbufs[b], tbufs[b])
            writeback(tbufs[b], osems[b], p)
        for b in range(_NBUF):
            drain_out(tbufs[b], osems[b])

    return k


def kernel(x, table):
    V, D = table.shape
    Bn, P = x.shape
    xt = x.T.astype(jnp.int32)
    out = _make_sc_gather(V, D, P, Bn)(table, xt)
    return out.reshape(P, D, Bn).transpose(2, 0, 1)


# speedup vs baseline: 2.7148x; 2.7148x over previous
import functools

import jax
import jax.numpy as jnp
from jax import lax
from jax.experimental import pallas as pl
from jax.experimental.pallas import tpu as pltpu
from jax.experimental.pallas import tpu_sc as plsc

SCALE = 8.0

_info = plsc.get_sparse_core_info()
_NC, _NS, _L = _info.num_cores, _info.num_subcores, _info.num_lanes
_NW = _NC * _NS

_NBUF = 2


@functools.lru_cache(maxsize=None)
def _make_sc_gather(V, D, P, N):
    C = N // _NW
    assert C <= 128 and D % _L == 0 and P % _NBUF == 0
    mesh = plsc.VectorSubcoreMesh(core_axis_name="c", subcore_axis_name="s")

    @functools.partial(
        pl.kernel,
        mesh=mesh,
        compiler_params=pltpu.CompilerParams(
            use_tc_tiling_on_sc=False, needs_layout_passes=False),
        out_type=jax.ShapeDtypeStruct((P, D // 8, N // 128, 8, 128),
                                      jnp.float32),
        scratch_types=[
            pltpu.VMEM((P, C), jnp.int32),
            pltpu.VMEM((C, D), jnp.float32),
            pltpu.VMEM((C, D), jnp.float32),
            pltpu.VMEM((D // 8, 8, C + 1), jnp.float32),
            pltpu.VMEM((D // 8, 8, C + 1), jnp.float32),
            pltpu.SemaphoreType.DMA,
            pltpu.SemaphoreType.DMA,
            pltpu.SemaphoreType.DMA,
            pltpu.SemaphoreType.DMA,
        ],
    )
    def k(table_hbm, idx_hbm, out_hbm, idx_v, g0, g1, t0, t1,
          gs0, gs1, os0, os1):
        gbufs, tbufs = (g0, g1), (t0, t1)
        gsems, osems = (gs0, gs1), (os0, os1)
        wid = lax.axis_index("s") * _NC + lax.axis_index("c")
        b0 = wid * C
        pltpu.sync_copy(idx_hbm.at[:, pl.ds(b0, C)], idx_v)

        @plsc.parallel_loop(0, P, unroll=4)
        def _dbl(r):
            for h in range(C // _L):
                sl = pl.ds(h * _L, _L)
                idx_v[r, sl] = idx_v[r, sl] * 2

        def fire(buf, sem, p):
            pltpu.async_copy(table_hbm.at[idx_v.at[p]], buf, sem)

        def drain_gather(buf, sem):
            pltpu.make_async_copy(table_hbm.at[idx_v.at[0]], buf, sem).wait()

        def transpose_scale(gbuf, tbuf):
            @plsc.parallel_loop(0, C, unroll=8)
            def body(r):
                cols = jnp.full((_L,), r, jnp.int32)
                for q in range(D // _L):
                    rows = lax.iota(jnp.int32, _L) + q * _L
                    v = gbuf[r, pl.ds(q * _L, _L)]
                    plsc.store_scatter(
                        tbuf, [rows // 8, rows % 8, cols], v * SCALE)

        def writeback(tbuf, sem, p):
            pltpu.async_copy(
                tbuf.at[:, :, pl.ds(0, C)],
                out_hbm.at[p, :, wid], sem)

        def drain_out(tbuf, sem):
            pltpu.make_async_copy(
                tbuf.at[:, :, pl.ds(0, C)],
                out_hbm.at[0, :, 0], sem).wait()

        for b in range(_NBUF):
            fire(gbufs[b], gsems[b], b)

        def outer(g, carry):
            for b in range(_NBUF):
                p = g * _NBUF + b
                drain_gather(gbufs[b], gsems[b])

                @pl.when(g > 0)
                def _():
                    drain_out(tbufs[b], osems[b])

                transpose_scale(gbufs[b], tbufs[b])
                fire(gbufs[b], gsems[b], p + _NBUF)
                writeback(tbufs[b], osems[b], p)
            return carry

        lax.fori_loop(0, P // _NBUF - 1, outer, 0)

        for b in range(_NBUF):
            p = P - _NBUF + b
            drain_gather(gbufs[b], gsems[b])
            drain_out(tbufs[b], osems[b])
            transpose_scale(gbufs[b], tbufs[b])
            writeback(tbufs[b], osems[b], p)
        for b in range(_NBUF):
            drain_out(tbufs[b], osems[b])

    return k


def kernel(x, table):
    V, D = table.shape
    Bn, P = x.shape
    xt = x.T.astype(jnp.int32)
    tpad = jnp.pad(table, ((0, 0), (0, D)))
    t2 = tpad.reshape(2 * V, D)
    out = _make_sc_gather(V, D, P, Bn)(t2, xt)
    return out.transpose(2, 4, 0, 1, 3).reshape(Bn, P, D)
